# Initial kernel scaffold; baseline (speedup 1.0000x reference)
import functools

import jax
import jax.numpy as jnp
from jax import lax
from jax.experimental import pallas as pl
from jax.experimental.pallas import tpu as pltpu
from jax.experimental.pallas import tpu_sc as plsc

_K = 64
_ROWS = 64
_N = 32768
_L = 16
_NB = _N // _L
_NC = 2
_NS = 16
_NW = _NC * _NS
_RPW = _ROWS // _NW

_mesh = plsc.VectorSubcoreMesh(
    core_axis_name="c", subcore_axis_name="s", num_cores=_NC, num_subcores=_NS)


def _keyify(v):
    s = v >> jnp.uint32(31)
    return v ^ ((jnp.uint32(0) - s) | jnp.uint32(0x80000000))


@functools.partial(
    pl.kernel,
    out_type=(
        jax.ShapeDtypeStruct((_ROWS, 8), jnp.uint32),
        jax.ShapeDtypeStruct((_ROWS, _K), jnp.int32),
        jax.ShapeDtypeStruct((_ROWS, _K), jnp.float32),
    ),
    mesh=_mesh,
    scratch_types=[
        pltpu.VMEM((_N,), jnp.uint32),
        pltpu.VMEM((_N + 2 * _L,), jnp.uint32),
        pltpu.VMEM((_N + 2 * _L,), jnp.int32),
        pltpu.VMEM((257 * _L,), jnp.int32),
        pltpu.VMEM((_K,), jnp.int32),
        pltpu.VMEM((_K,), jnp.float32),
        pltpu.VMEM((_L,), jnp.uint32),
    ],
)
def _sc_select(x_ref, tkey_ref, tidx_ref, tval_ref,
               row_buf, ckey, cidx, hist, stage_i, stage_f, stage_k):
    wid = lax.axis_index("s") * _NC + lax.axis_index("c")
    lane = lax.broadcasted_iota(jnp.int32, (_L,), 0)
    ones = jnp.ones((_L,), jnp.int32)
    zeros16 = jnp.zeros((_L,), jnp.int32)

    def zero_hist():
        def zb(i, _):
            hist[pl.ds(i * _L, _L)] = zeros16
            return 0
        lax.fori_loop(0, 257, zb, 0)

    def find_bucket(r):
        def sb(j, acc):
            b = 255 - j
            acc = acc + hist[pl.ds(b * _L, _L)]
            hist[pl.ds(b * _L, _L)] = acc
            return acc
        lax.fori_loop(0, 256, sb, zeros16)
        bkt = jnp.int32(0)
        for step in (128, 64, 32, 16, 8, 4, 2, 1):
            cand = bkt + step
            s = jnp.sum(hist[pl.ds(cand * _L, _L)])
            bkt = jnp.where(s >= r, cand, bkt)
        above = jnp.sum(hist[pl.ds((bkt + 1) * _L, _L)])
        return bkt, r - above

    for rr in range(_RPW):
        row = wid * _RPW + rr
        pltpu.sync_copy(x_ref.at[row], row_buf)

        zero_hist()

        def p1(i, _):
            k = _keyify(row_buf[pl.ds(i * _L, _L)])
            b = lax.convert_element_type(k >> jnp.uint32(24), jnp.int32)
            plsc.addupdate_scatter(hist, [(b << 4) + lane], ones)
            return 0
        lax.fori_loop(0, _NB, p1, 0)
        b1, r = find_bucket(jnp.int32(_K))
        b1u = lax.convert_element_type(b1, jnp.uint32)

        zero_hist()
        gbase = row * _N

        def p2(i, w):
            k = _keyify(row_buf[pl.ds(i * _L, _L)])
            match = (k >> jnp.uint32(24)) == b1u
            b = lax.convert_element_type((k >> jnp.uint32(16)) & jnp.uint32(0xFF),
                                         jnp.int32)
            plsc.addupdate_scatter(hist, [(b << 4) + lane], ones, mask=match)
            plsc.store_compressed(ckey.at[pl.ds(w, _L)], k, mask=match)
            g = (gbase + i * _L) + lane
            plsc.store_compressed(cidx.at[pl.ds(w, _L)], g, mask=match)
            return w + jnp.sum(lax.convert_element_type(match, jnp.int32))
        m = lax.fori_loop(0, _NB, p2, jnp.int32(0))
        b2, r = find_bucket(r)

        def refine(m, r, bp, sp, sn):
            bpu = lax.convert_element_type(bp, jnp.uint32)
            zero_hist()
            nb = (m + (_L - 1)) // _L

            def pp(i, w):
                k = ckey[pl.ds(i * _L, _L)]
                g = cidx[pl.ds(i * _L, _L)]
                valid = (i * _L + lane) < m
                match = valid & (((k >> jnp.uint32(sp)) & jnp.uint32(0xFF)) == bpu)
                b = lax.convert_element_type((k >> jnp.uint32(sn)) & jnp.uint32(0xFF),
                                             jnp.int32)
                plsc.addupdate_scatter(hist, [(b << 4) + lane], ones, mask=match)
                plsc.store_compressed(ckey.at[pl.ds(w, _L)], k, mask=match)
                plsc.store_compressed(cidx.at[pl.ds(w, _L)], g, mask=match)
                return w + jnp.sum(lax.convert_element_type(match, jnp.int32))
            m2 = lax.fori_loop(0, nb, pp, jnp.int32(0))
            bn, r2 = find_bucket(r)
            return m2, r2, bn

        m, r, b3 = refine(m, r, b2, 16, 8)
        m, r, b4 = refine(m, r, b3, 8, 0)

        b2u = lax.convert_element_type(b2, jnp.uint32)
        b3u = lax.convert_element_type(b3, jnp.uint32)
        b4u = lax.convert_element_type(b4, jnp.uint32)
        t_key = ((b1u << jnp.uint32(24)) | (b2u << jnp.uint32(16))
                 | (b3u << jnp.uint32(8)) | b4u)

        nb_t = (m + (_L - 1)) // _L

        def tp(i, w):
            k = ckey[pl.ds(i * _L, _L)]
            g = cidx[pl.ds(i * _L, _L)]
            valid = (i * _L + lane) < m
            eq = valid & (k == t_key)
            plsc.store_compressed(cidx.at[pl.ds(w, _L)], g, mask=eq)
            return w + jnp.sum(lax.convert_element_type(eq, jnp.int32))
        lax.fori_loop(0, nb_t, tp, jnp.int32(0))

        s0 = cidx[0]
        for j in range(_K // _L):
            lane_e = j * _L + lane
            v = cidx[pl.ds(j * _L, _L)]
            stage_i[pl.ds(j * _L, _L)] = jnp.where(lane_e < r, v,
                                                   jnp.full((_L,), s0, jnp.int32))
        pltpu.sync_copy(stage_i, tidx_ref.at[row])

        tkv = jnp.full((_L,), t_key)
        sgn = tkv >> jnp.uint32(31)
        obits = tkv ^ ((jnp.uint32(0) - (jnp.uint32(1) - sgn))
                       | jnp.uint32(0x80000000))
        tvalv = plsc.bitcast(obits, jnp.float32)
        for j in range(_K // _L):
            stage_f[pl.ds(j * _L, _L)] = tvalv
        pltpu.sync_copy(stage_f, tval_ref.at[row])

        stage_k[pl.ds(0, _L)] = tkv
        pltpu.sync_copy(stage_k.at[pl.ds(0, 8)], tkey_ref.at[row])


def _tc_mask_body(x_ref, t_ref, o_ref):
    v = x_ref[...]
    k = _keyify(v)
    t = t_ref[:, 0:1]
    o_ref[...] = jnp.where(k > t, lax.bitcast_convert_type(v, jnp.float32),
                           jnp.float32(0.0))


_TC_BLK = 4096
_tc_mask = pl.pallas_call(
    _tc_mask_body,
    grid=(_N // _TC_BLK,),
    in_specs=[
        pl.BlockSpec((_ROWS, _TC_BLK), lambda j: (0, j)),
        pl.BlockSpec((_ROWS, 8), lambda j: (0, 0)),
    ],
    out_specs=pl.BlockSpec((_ROWS, _TC_BLK), lambda j: (0, j)),
    out_shape=jax.ShapeDtypeStruct((_ROWS, _N), jnp.float32),
)


@functools.partial(
    pl.kernel,
    out_type=(),
    mesh=_mesh,
    scratch_types=[
        pltpu.VMEM((_K,), jnp.int32),
        pltpu.VMEM((_K,), jnp.float32),
        pltpu.SemaphoreType.DMA,
    ],
)
def _sc_fixup(out_ref, gidx_ref, val_ref, idx_v, val_v, sem):
    wid = lax.axis_index("s") * _NC + lax.axis_index("c")
    for rr in range(_RPW):
        row = wid * _RPW + rr
        pltpu.sync_copy(gidx_ref.at[row], idx_v)
        pltpu.sync_copy(val_ref.at[row], val_v)
        pltpu.async_copy(val_v, out_ref.at[idx_v], sem).wait()


def kernel(x):
    xi = lax.bitcast_convert_type(x, jnp.uint32)
    tkeys, tidx, tvals = _sc_select(xi)
    out = _tc_mask(xi, tkeys)
    out_ref = jax.new_ref(out.reshape(_ROWS * _N))
    _sc_fixup(out_ref, tidx, tvals)
    return out_ref[...].reshape(_ROWS, _N)


# speedup vs baseline: 2.2428x; 2.2428x over previous
import functools

import jax
import jax.numpy as jnp
from jax import lax
from jax.experimental import pallas as pl
from jax.experimental.pallas import tpu as pltpu
from jax.experimental.pallas import tpu_sc as plsc

_K = 64
_ROWS = 64
_N = 32768
_L = 16
_NB = _N // _L
_NC = 2
_NS = 16
_NW = _NC * _NS
_RPW = _ROWS // _NW

_mesh = plsc.VectorSubcoreMesh(
    core_axis_name="c", subcore_axis_name="s", num_cores=_NC, num_subcores=_NS)


def _keyify(v):
    s = v >> jnp.uint32(31)
    return v ^ ((jnp.uint32(0) - s) | jnp.uint32(0x80000000))


@functools.partial(
    pl.kernel,
    out_type=(
        jax.ShapeDtypeStruct((_ROWS, _K), jnp.uint32),
        jax.ShapeDtypeStruct((_ROWS, _K), jnp.int32),
        jax.ShapeDtypeStruct((_ROWS, _K), jnp.float32),
    ),
    mesh=_mesh,
    scratch_types=[
        pltpu.VMEM((_N,), jnp.uint32),
        pltpu.VMEM((_N + 2 * _L,), jnp.uint32),
        pltpu.VMEM((_N + 2 * _L,), jnp.int32),
        pltpu.VMEM((257 * _L,), jnp.int32),
        pltpu.VMEM((_K,), jnp.int32),
        pltpu.VMEM((_K,), jnp.float32),
        pltpu.VMEM((_K,), jnp.uint32),
    ],
    compiler_params=pltpu.CompilerParams(needs_layout_passes=False),
)
def _sc_select(x_ref, tkey_ref, tidx_ref, tval_ref,
               row_buf, ckey, cidx, hist, stage_i, stage_f, stage_k):
    wid = lax.axis_index("s") * _NC + lax.axis_index("c")
    lane = lax.broadcasted_iota(jnp.int32, (_L,), 0)
    ones = jnp.ones((_L,), jnp.int32)
    zeros16 = jnp.zeros((_L,), jnp.int32)

    def zero_hist():
        def zb(i, _):
            hist[pl.ds(i * _L, _L)] = zeros16
            return 0
        lax.fori_loop(0, 257, zb, 0)

    def find_bucket(r):
        def sb(j, acc):
            b = 255 - j
            acc = acc + hist[pl.ds(b * _L, _L)]
            hist[pl.ds(b * _L, _L)] = acc
            return acc
        lax.fori_loop(0, 256, sb, zeros16)
        bkt = jnp.int32(0)
        for step in (128, 64, 32, 16, 8, 4, 2, 1):
            cand = bkt + step
            s = jnp.sum(hist[pl.ds(cand * _L, _L)])
            bkt = jnp.where(s >= r, cand, bkt)
        above = jnp.sum(hist[pl.ds((bkt + 1) * _L, _L)])
        return bkt, r - above

    for rr in range(_RPW):
        row = wid * _RPW + rr
        pltpu.sync_copy(x_ref.at[row], row_buf)

        zero_hist()

        def p1(i, _):
            k = _keyify(row_buf[pl.ds(i * _L, _L)])
            b = lax.convert_element_type(k >> jnp.uint32(24), jnp.int32)
            plsc.addupdate_scatter(hist, [(b << 4) + lane], ones)
            return 0
        lax.fori_loop(0, _NB, p1, 0)
        b1, r = find_bucket(jnp.int32(_K))
        b1u = lax.convert_element_type(b1, jnp.uint32)

        zero_hist()
        gbase = row * _N

        def p2(i, w):
            k = _keyify(row_buf[pl.ds(i * _L, _L)])
            match = (k >> jnp.uint32(24)) == b1u
            b = lax.convert_element_type((k >> jnp.uint32(16)) & jnp.uint32(0xFF),
                                         jnp.int32)
            plsc.addupdate_scatter(hist, [(b << 4) + lane], ones, mask=match)
            plsc.store_compressed(ckey.at[pl.ds(w, _L)], k, mask=match)
            g = (gbase + i * _L) + lane
            plsc.store_compressed(cidx.at[pl.ds(w, _L)], g, mask=match)
            return w + jnp.sum(lax.convert_element_type(match, jnp.int32))
        m = lax.fori_loop(0, _NB, p2, jnp.int32(0))
        b2, r = find_bucket(r)

        def refine(m, r, bp, sp, sn):
            bpu = lax.convert_element_type(bp, jnp.uint32)
            zero_hist()
            nb = (m + (_L - 1)) // _L

            def pp(i, w):
                k = ckey[pl.ds(i * _L, _L)]
                g = cidx[pl.ds(i * _L, _L)]
                valid = (i * _L + lane) < m
                match = valid & (((k >> jnp.uint32(sp)) & jnp.uint32(0xFF)) == bpu)
                b = lax.convert_element_type((k >> jnp.uint32(sn)) & jnp.uint32(0xFF),
                                             jnp.int32)
                plsc.addupdate_scatter(hist, [(b << 4) + lane], ones, mask=match)
                plsc.store_compressed(ckey.at[pl.ds(w, _L)], k, mask=match)
                plsc.store_compressed(cidx.at[pl.ds(w, _L)], g, mask=match)
                return w + jnp.sum(lax.convert_element_type(match, jnp.int32))
            m2 = lax.fori_loop(0, nb, pp, jnp.int32(0))
            bn, r2 = find_bucket(r)
            return m2, r2, bn

        m, r, b3 = refine(m, r, b2, 16, 8)
        m, r, b4 = refine(m, r, b3, 8, 0)

        b2u = lax.convert_element_type(b2, jnp.uint32)
        b3u = lax.convert_element_type(b3, jnp.uint32)
        b4u = lax.convert_element_type(b4, jnp.uint32)
        t_key = ((b1u << jnp.uint32(24)) | (b2u << jnp.uint32(16))
                 | (b3u << jnp.uint32(8)) | b4u)

        nb_t = (m + (_L - 1)) // _L

        def tp(i, w):
            k = ckey[pl.ds(i * _L, _L)]
            g = cidx[pl.ds(i * _L, _L)]
            valid = (i * _L + lane) < m
            eq = valid & (k == t_key)
            plsc.store_compressed(cidx.at[pl.ds(w, _L)], g, mask=eq)
            return w + jnp.sum(lax.convert_element_type(eq, jnp.int32))
        lax.fori_loop(0, nb_t, tp, jnp.int32(0))

        s0 = cidx[pl.ds(0, _L)][0]
        for j in range(_K // _L):
            lane_e = j * _L + lane
            v = cidx[pl.ds(j * _L, _L)]
            stage_i[pl.ds(j * _L, _L)] = jnp.where(lane_e < r, v,
                                                   jnp.full((_L,), s0, jnp.int32))
        pltpu.sync_copy(stage_i, tidx_ref.at[row])

        tkv = jnp.full((_L,), t_key)
        sgn = tkv >> jnp.uint32(31)
        obits = tkv ^ ((jnp.uint32(0) - (jnp.uint32(1) - sgn))
                       | jnp.uint32(0x80000000))
        tvalv = plsc.bitcast(obits, jnp.float32)
        for j in range(_K // _L):
            stage_f[pl.ds(j * _L, _L)] = tvalv
        pltpu.sync_copy(stage_f, tval_ref.at[row])

        for j in range(_K // _L):
            stage_k[pl.ds(j * _L, _L)] = tkv
        pltpu.sync_copy(stage_k, tkey_ref.at[row])


def _tc_mask_body(x_ref, t_ref, o_ref):
    v = x_ref[...]
    k = _keyify(v)
    t = t_ref[:, 0:1]
    o_ref[...] = jnp.where(k > t, lax.bitcast_convert_type(v, jnp.float32),
                           jnp.float32(0.0))


_TC_BLK = 4096
_tc_mask = pl.pallas_call(
    _tc_mask_body,
    grid=(_N // _TC_BLK,),
    in_specs=[
        pl.BlockSpec((_ROWS, _TC_BLK), lambda j: (0, j)),
        pl.BlockSpec((_ROWS, _K), lambda j: (0, 0)),
    ],
    out_specs=pl.BlockSpec((_ROWS, _TC_BLK), lambda j: (0, j)),
    out_shape=jax.ShapeDtypeStruct((_ROWS, _N), jnp.float32),
)


@functools.partial(
    pl.kernel,
    out_type=(),
    mesh=_mesh,
    scratch_types=[
        pltpu.VMEM((_K,), jnp.int32),
        pltpu.VMEM((_K,), jnp.float32),
        pltpu.SemaphoreType.DMA,
    ],
)
def _sc_fixup(out_ref, gidx_ref, val_ref, idx_v, val_v, sem):
    wid = lax.axis_index("s") * _NC + lax.axis_index("c")
    for rr in range(_RPW):
        row = wid * _RPW + rr
        pltpu.sync_copy(gidx_ref.at[row], idx_v)
        pltpu.sync_copy(val_ref.at[row], val_v)
        pltpu.async_copy(val_v, out_ref.at[idx_v], sem).wait()


def kernel(x):
    xi = lax.bitcast_convert_type(x, jnp.uint32)
    tkeys, tidx, tvals = _sc_select(xi)
    out = _tc_mask(xi, tkeys)
    out_ref = jax.new_ref(out.reshape(_ROWS * _N))
    _sc_fixup(out_ref, tidx, tvals)
    return out_ref[...].reshape(_ROWS, _N)


# speedup vs baseline: 2.5504x; 1.1372x over previous
import functools

import jax
import jax.numpy as jnp
import numpy as np
from jax import lax
from jax.experimental import pallas as pl
from jax.experimental.pallas import tpu as pltpu
from jax.experimental.pallas import tpu_sc as plsc

_K = 64
_ROWS = 64
_N = 32768
_L = 16
_NB = _N // _L
_NC = 2
_NS = 16
_NW = _NC * _NS
_RPW = _ROWS // _NW
_U = 4

_mesh = plsc.VectorSubcoreMesh(
    core_axis_name="c", subcore_axis_name="s", num_cores=_NC, num_subcores=_NS)


def _keyify(v):
    s = v >> jnp.uint32(31)
    return v ^ ((jnp.uint32(0) - s) | jnp.uint32(0x80000000))


@functools.partial(
    pl.kernel,
    out_type=jax.ShapeDtypeStruct((_ROWS, _K), jnp.uint32),
    mesh=_mesh,
    scratch_types=[
        pltpu.VMEM((_N,), jnp.uint32),
        pltpu.VMEM((_N + 2 * _L,), jnp.int32),
        pltpu.VMEM((264 * _L,), jnp.int32),
        pltpu.VMEM((_K,), jnp.uint32),
    ],
    compiler_params=pltpu.CompilerParams(needs_layout_passes=False),
)
def _sc_select(x_ref, tkey_ref, row_buf, ckey, hist, stage_k):
    wid = lax.axis_index("s") * _NC + lax.axis_index("c")
    lane = lax.broadcasted_iota(jnp.int32, (_L,), 0)
    ones = jnp.ones((_L,), jnp.int32)
    zeros16 = jnp.zeros((_L,), jnp.int32)

    def zero_hist():
        def zb(i, _):
            for u in range(8):
                hist[pl.ds((i * 8 + u) * _L, _L)] = zeros16
            return 0
        lax.fori_loop(0, 33, zb, 0)

    def find_bucket(r):
        def sb(j, acc):
            for u in range(4):
                b = 255 - (j * 4 + u)
                acc = acc + hist[pl.ds(b * _L, _L)]
                hist[pl.ds(b * _L, _L)] = acc
            return acc
        lax.fori_loop(0, 64, sb, zeros16)
        bkt = jnp.int32(0)
        for step in (128, 64, 32, 16, 8, 4, 2, 1):
            cand = bkt + step
            s = jnp.sum(hist[pl.ds(cand * _L, _L)])
            bkt = jnp.where(s >= r, cand, bkt)
        above = jnp.sum(hist[pl.ds((bkt + 1) * _L, _L)])
        return bkt, r - above

    for rr in range(_RPW):
        row = wid * _RPW + rr
        pltpu.sync_copy(x_ref.at[row], row_buf)

        zero_hist()

        def p1(i, _):
            for u in range(_U):
                k = _keyify(row_buf[pl.ds((i * _U + u) * _L, _L)])
                b = lax.convert_element_type(k >> jnp.uint32(24), jnp.int32)
                plsc.addupdate_scatter(hist, [(b << 4) + lane], ones)
            return 0
        lax.fori_loop(0, _NB // _U, p1, 0)
        b1, r = find_bucket(jnp.int32(_K))
        b1u = lax.convert_element_type(b1, jnp.uint32)

        zero_hist()

        def p2(i, w):
            for u in range(_U):
                k = _keyify(row_buf[pl.ds((i * _U + u) * _L, _L)])
                match = (k >> jnp.uint32(24)) == b1u
                b = lax.convert_element_type(
                    (k >> jnp.uint32(16)) & jnp.uint32(0xFF), jnp.int32)
                plsc.addupdate_scatter(hist, [(b << 4) + lane], ones, mask=match)
                mi = lax.convert_element_type(match, jnp.int32)
                pos = w + plsc.cumsum(mi) - 1
                plsc.store_scatter(ckey, [pos], plsc.bitcast(k, jnp.int32),
                                   mask=match)
                w = w + plsc.all_reduce_population_count(match)
            return w
        w_vec = lax.fori_loop(0, _NB // _U, p2, zeros16)
        m = w_vec[0]
        b2, r = find_bucket(r)

        def refine(m, r, bp, sp, sn):
            bpu = lax.convert_element_type(bp, jnp.uint32)
            zero_hist()
            nb = (m + (_U * _L - 1)) // (_U * _L)

            def pp(i, w):
                for u in range(_U):
                    e0 = (i * _U + u) * _L
                    k = plsc.bitcast(ckey[pl.ds(e0, _L)], jnp.uint32)
                    valid = (e0 + lane) < m
                    match = valid & (
                        ((k >> jnp.uint32(sp)) & jnp.uint32(0xFF)) == bpu)
                    b = lax.convert_element_type(
                        (k >> jnp.uint32(sn)) & jnp.uint32(0xFF), jnp.int32)
                    plsc.addupdate_scatter(hist, [(b << 4) + lane], ones,
                                           mask=match)
                    mi = lax.convert_element_type(match, jnp.int32)
                    pos = w + plsc.cumsum(mi) - 1
                    plsc.store_scatter(ckey, [pos], plsc.bitcast(k, jnp.int32),
                                       mask=match)
                    w = w + plsc.all_reduce_population_count(match)
                return w
            w2 = lax.fori_loop(0, nb, pp, zeros16)
            bn, r2 = find_bucket(r)
            return w2[0], r2, bn

        m, r, b3 = refine(m, r, b2, 16, 8)
        m, r, b4 = refine(m, r, b3, 8, 0)

        b2u = lax.convert_element_type(b2, jnp.uint32)
        b3u = lax.convert_element_type(b3, jnp.uint32)
        b4u = lax.convert_element_type(b4, jnp.uint32)
        t_key = ((b1u << jnp.uint32(24)) | (b2u << jnp.uint32(16))
                 | (b3u << jnp.uint32(8)) | b4u)
        e_take = lax.convert_element_type(r, jnp.uint32)

        tkv = jnp.where(lane == 1, jnp.full((_L,), e_take),
                        jnp.full((_L,), t_key))
        for j in range(_K // _L):
            stage_k[pl.ds(j * _L, _L)] = tkv
        pltpu.sync_copy(stage_k, tkey_ref.at[row])


_TC_BLK = 512
_TRI = np.triu(np.ones((_TC_BLK, _TC_BLK), np.float32))


def _tc_mask_body(x_ref, t_ref, tri_ref, o_ref, cnt_ref):
    @pl.when(pl.program_id(0) == 0)
    def _():
        cnt_ref[...] = jnp.zeros_like(cnt_ref)

    v = x_ref[...]
    k = _keyify(v)
    t = t_ref[:, 0:1]
    e_take = lax.convert_element_type(t_ref[:, 1:2], jnp.float32)
    gt = k > t
    eq = k == t
    eqb = lax.convert_element_type(eq, jnp.bfloat16)
    csum = jax.lax.dot_general(
        eqb, tri_ref[...], (((1,), (0,)), ((), ())),
        preferred_element_type=jnp.float32)
    rank = csum + cnt_ref[:, 0:1]
    take = eq & (rank <= e_take)
    o_ref[...] = jnp.where(gt | take,
                           lax.bitcast_convert_type(v, jnp.float32),
                           jnp.float32(0.0))
    cnt_ref[:, 0:1] = cnt_ref[:, 0:1] + jnp.sum(
        lax.convert_element_type(eq, jnp.float32), axis=1, keepdims=True)


_tc_mask = pl.pallas_call(
    _tc_mask_body,
    grid=(_N // _TC_BLK,),
    in_specs=[
        pl.BlockSpec((_ROWS, _TC_BLK), lambda j: (0, j)),
        pl.BlockSpec((_ROWS, _K), lambda j: (0, 0)),
        pl.BlockSpec((_TC_BLK, _TC_BLK), lambda j: (0, 0)),
    ],
    out_specs=pl.BlockSpec((_ROWS, _TC_BLK), lambda j: (0, j)),
    out_shape=jax.ShapeDtypeStruct((_ROWS, _N), jnp.float32),
    scratch_shapes=[pltpu.VMEM((_ROWS, 128), jnp.float32)],
)


def kernel(x):
    xi = lax.bitcast_convert_type(x, jnp.uint32)
    tkeys = _sc_select(xi)
    tri = jnp.asarray(_TRI, jnp.bfloat16)
    return _tc_mask(xi, tkeys, tri)


# speedup vs baseline: 4.9122x; 1.9260x over previous
import functools

import jax
import jax.numpy as jnp
import numpy as np
from jax import lax
from jax.experimental import pallas as pl
from jax.experimental.pallas import tpu as pltpu
from jax.experimental.pallas import tpu_sc as plsc

_K = 64
_ROWS = 64
_N = 32768
_L = 16
_NB = _N // _L
_NC = 2
_NS = 16
_NW = _NC * _NS
_RPW = _ROWS // _NW
_U = 4

_mesh = plsc.VectorSubcoreMesh(
    core_axis_name="c", subcore_axis_name="s", num_cores=_NC, num_subcores=_NS)


def _keyify(v):
    s = v >> jnp.uint32(31)
    return v ^ ((jnp.uint32(0) - s) | jnp.uint32(0x80000000))


@functools.partial(
    pl.kernel,
    out_type=jax.ShapeDtypeStruct((_ROWS, _K), jnp.uint32),
    mesh=_mesh,
    scratch_types=[
        pltpu.VMEM((_N,), jnp.float32),
        pltpu.VMEM((_N + 2 * _L,), jnp.int32),
        pltpu.VMEM((264 * _L,), jnp.int32),
        pltpu.VMEM((_K,), jnp.uint32),
    ],
    compiler_params=pltpu.CompilerParams(needs_layout_passes=False),
)
def _sc_select(x_ref, tkey_ref, row_buf, ckey, hist, stage_k):
    wid = lax.axis_index("s") * _NC + lax.axis_index("c")
    lane = lax.broadcasted_iota(jnp.int32, (_L,), 0)
    ones = jnp.ones((_L,), jnp.int32)
    zeros16 = jnp.zeros((_L,), jnp.int32)

    def zero_hist():
        @plsc.parallel_loop(0, 264, unroll=8)
        def _(i):
            hist[pl.ds(i * _L, _L)] = zeros16

    def find_bucket(r):
        @plsc.parallel_loop(0, 256, unroll=4, carry=zeros16)
        def _sfx(j, acc):
            b = 255 - j
            acc = acc + hist[pl.ds(b * _L, _L)]
            hist[pl.ds(b * _L, _L)] = acc
            return acc
        bkt = jnp.int32(0)
        for step in (128, 64, 32, 16, 8, 4, 2, 1):
            cand = bkt + step
            s = jnp.sum(hist[pl.ds(cand * _L, _L)])
            bkt = jnp.where(s >= r, cand, bkt)
        above = jnp.sum(hist[pl.ds((bkt + 1) * _L, _L)])
        return bkt, r - above

    for rr in range(_RPW):
        row = wid * _RPW + rr
        pltpu.sync_copy(x_ref.at[row], row_buf)

        zero_hist()

        @plsc.parallel_loop(0, _NB, unroll=8)
        def _p1(i):
            k = _keyify(plsc.bitcast(row_buf[pl.ds(i * _L, _L)], jnp.uint32))
            b = lax.convert_element_type(k >> jnp.uint32(24), jnp.int32)
            plsc.addupdate_scatter(hist, [(b << 4) + lane], ones)
        b1, r = find_bucket(jnp.int32(_K))
        b1u = lax.convert_element_type(b1, jnp.uint32)

        zero_hist()

        @plsc.parallel_loop(0, _NB, unroll=4, carry=zeros16)
        def _p2(i, w):
            k = _keyify(plsc.bitcast(row_buf[pl.ds(i * _L, _L)], jnp.uint32))
            match = (k >> jnp.uint32(24)) == b1u
            b = lax.convert_element_type(
                (k >> jnp.uint32(16)) & jnp.uint32(0xFF), jnp.int32)
            plsc.addupdate_scatter(hist, [(b << 4) + lane], ones, mask=match)
            mi = lax.convert_element_type(match, jnp.int32)
            pos = w + plsc.cumsum(mi) - 1
            plsc.store_scatter(ckey, [pos], plsc.bitcast(k, jnp.int32),
                               mask=match)
            return w + plsc.all_reduce_population_count(match)
        m = _p2[0]
        b2, r = find_bucket(r)

        def refine(m, r, bp, sp, sn):
            bpu = lax.convert_element_type(bp, jnp.uint32)
            zero_hist()
            nb = (m + (_U * _L - 1)) // (_U * _L)

            def pp(i, w):
                for u in range(_U):
                    e0 = (i * _U + u) * _L
                    k = plsc.bitcast(ckey[pl.ds(e0, _L)], jnp.uint32)
                    valid = (e0 + lane) < m
                    match = valid & (
                        ((k >> jnp.uint32(sp)) & jnp.uint32(0xFF)) == bpu)
                    b = lax.convert_element_type(
                        (k >> jnp.uint32(sn)) & jnp.uint32(0xFF), jnp.int32)
                    plsc.addupdate_scatter(hist, [(b << 4) + lane], ones,
                                           mask=match)
                    mi = lax.convert_element_type(match, jnp.int32)
                    pos = w + plsc.cumsum(mi) - 1
                    plsc.store_scatter(ckey, [pos], plsc.bitcast(k, jnp.int32),
                                       mask=match)
                    w = w + plsc.all_reduce_population_count(match)
                return w
            w2 = lax.fori_loop(0, nb, pp, zeros16)
            bn, r2 = find_bucket(r)
            return w2[0], r2, bn

        m, r, b3 = refine(m, r, b2, 16, 8)
        m, r, b4 = refine(m, r, b3, 8, 0)

        b2u = lax.convert_element_type(b2, jnp.uint32)
        b3u = lax.convert_element_type(b3, jnp.uint32)
        b4u = lax.convert_element_type(b4, jnp.uint32)
        t_key = ((b1u << jnp.uint32(24)) | (b2u << jnp.uint32(16))
                 | (b3u << jnp.uint32(8)) | b4u)
        e_take = lax.convert_element_type(r, jnp.uint32)

        tkv = jnp.where(lane == 1, jnp.full((_L,), e_take),
                        jnp.full((_L,), t_key))
        for j in range(_K // _L):
            stage_k[pl.ds(j * _L, _L)] = tkv
        pltpu.sync_copy(stage_k, tkey_ref.at[row])


_TC_BLK = 512
_TRI = np.triu(np.ones((_TC_BLK, _TC_BLK), np.float32))


def _tc_mask_body(x_ref, t_ref, tri_ref, o_ref, cnt_ref):
    @pl.when(pl.program_id(0) == 0)
    def _():
        cnt_ref[...] = jnp.zeros_like(cnt_ref)

    vf = x_ref[...]
    v = lax.bitcast_convert_type(vf, jnp.uint32)
    k = _keyify(v)
    t = t_ref[:, 0:1]
    e_take = lax.convert_element_type(t_ref[:, 1:2], jnp.float32)
    gt = k > t
    eq = k == t
    eqb = lax.convert_element_type(eq, jnp.bfloat16)
    csum = jax.lax.dot_general(
        eqb, tri_ref[...], (((1,), (0,)), ((), ())),
        preferred_element_type=jnp.float32)
    rank = csum + cnt_ref[:, 0:1]
    take = eq & (rank <= e_take)
    o_ref[...] = jnp.where(gt | take, vf, jnp.float32(0.0))
    cnt_ref[:, 0:1] = cnt_ref[:, 0:1] + jnp.sum(
        lax.convert_element_type(eq, jnp.float32), axis=1, keepdims=True)


_tc_mask = pl.pallas_call(
    _tc_mask_body,
    grid=(_N // _TC_BLK,),
    in_specs=[
        pl.BlockSpec((_ROWS, _TC_BLK), lambda j: (0, j)),
        pl.BlockSpec((_ROWS, _K), lambda j: (0, 0)),
        pl.BlockSpec((_TC_BLK, _TC_BLK), lambda j: (0, 0)),
    ],
    out_specs=pl.BlockSpec((_ROWS, _TC_BLK), lambda j: (0, j)),
    out_shape=jax.ShapeDtypeStruct((_ROWS, _N), jnp.float32),
    scratch_shapes=[pltpu.VMEM((_ROWS, 128), jnp.float32)],
)


def kernel(x):
    tkeys = _sc_select(x)
    tri = jnp.asarray(_TRI, jnp.bfloat16)
    return _tc_mask(x, tkeys, tri)


# speedup vs baseline: 7.1781x; 1.4613x over previous
import functools

import jax
import jax.numpy as jnp
from jax import lax
from jax.experimental import pallas as pl
from jax.experimental.pallas import tpu as pltpu
from jax.experimental.pallas import tpu_sc as plsc

_K = 64
_ROWS = 64
_N = 32768
_L = 16
_NB = _N // _L
_NC = 2
_NS = 16
_NW = _NC * _NS
_RPW = _ROWS // _NW
_U = 4

_mesh = plsc.VectorSubcoreMesh(
    core_axis_name="c", subcore_axis_name="s", num_cores=_NC, num_subcores=_NS)


def _keyify(v):
    s = v >> jnp.uint32(31)
    return v ^ ((jnp.uint32(0) - s) | jnp.uint32(0x80000000))


@functools.partial(
    pl.kernel,
    out_type=jax.ShapeDtypeStruct((_ROWS, _N), jnp.float32),
    mesh=_mesh,
    scratch_types=[
        pltpu.VMEM((_N,), jnp.float32),
        pltpu.VMEM((_N,), jnp.float32),
        pltpu.VMEM((_N + 2 * _L,), jnp.int32),
        pltpu.VMEM((264 * _L,), jnp.int32),
        pltpu.SemaphoreType.DMA,
    ],
    compiler_params=pltpu.CompilerParams(needs_layout_passes=False),
)
def _sc_topk(x_ref, o_ref, row_buf, out_buf, ckey, hist, sem):
    wid = lax.axis_index("s") * _NC + lax.axis_index("c")
    lane = lax.broadcasted_iota(jnp.int32, (_L,), 0)
    ones = jnp.ones((_L,), jnp.int32)
    zeros16 = jnp.zeros((_L,), jnp.int32)

    def zero_hist():
        @plsc.parallel_loop(0, 264, unroll=8)
        def _(i):
            hist[pl.ds(i * _L, _L)] = zeros16

    def find_bucket(r):
        @plsc.parallel_loop(0, 256, unroll=4, carry=zeros16)
        def _sfx(j, acc):
            b = 255 - j
            acc = acc + hist[pl.ds(b * _L, _L)]
            hist[pl.ds(b * _L, _L)] = acc
            return acc
        bkt = jnp.int32(0)
        for step in (128, 64, 32, 16, 8, 4, 2, 1):
            cand = bkt + step
            s = jnp.sum(hist[pl.ds(cand * _L, _L)])
            bkt = jnp.where(s >= r, cand, bkt)
        above = jnp.sum(hist[pl.ds((bkt + 1) * _L, _L)])
        return bkt, r - above

    out_dma = None
    for rr in range(_RPW):
        row = wid * _RPW + rr
        pltpu.sync_copy(x_ref.at[row], row_buf)

        zero_hist()

        @plsc.parallel_loop(0, _NB, unroll=8)
        def _p1(i):
            k = _keyify(plsc.bitcast(row_buf[pl.ds(i * _L, _L)], jnp.uint32))
            b = lax.convert_element_type(k >> jnp.uint32(24), jnp.int32)
            plsc.addupdate_scatter(hist, [(b << 4) + lane], ones)
        b1, r = find_bucket(jnp.int32(_K))
        b1u = lax.convert_element_type(b1, jnp.uint32)

        zero_hist()

        @plsc.parallel_loop(0, _NB, unroll=4, carry=zeros16)
        def _p2(i, w):
            k = _keyify(plsc.bitcast(row_buf[pl.ds(i * _L, _L)], jnp.uint32))
            match = (k >> jnp.uint32(24)) == b1u
            b = lax.convert_element_type(
                (k >> jnp.uint32(16)) & jnp.uint32(0xFF), jnp.int32)
            plsc.addupdate_scatter(hist, [(b << 4) + lane], ones, mask=match)
            mi = lax.convert_element_type(match, jnp.int32)
            pos = w + plsc.cumsum(mi) - 1
            plsc.store_scatter(ckey, [pos], plsc.bitcast(k, jnp.int32),
                               mask=match)
            return w + plsc.all_reduce_population_count(match)
        m = _p2[0]
        b2, r = find_bucket(r)

        def refine(m, r, bp, sp, sn):
            bpu = lax.convert_element_type(bp, jnp.uint32)
            zero_hist()
            nb = (m + (_U * _L - 1)) // (_U * _L)

            def pp(i, w):
                for u in range(_U):
                    e0 = (i * _U + u) * _L
                    k = plsc.bitcast(ckey[pl.ds(e0, _L)], jnp.uint32)
                    valid = (e0 + lane) < m
                    match = valid & (
                        ((k >> jnp.uint32(sp)) & jnp.uint32(0xFF)) == bpu)
                    b = lax.convert_element_type(
                        (k >> jnp.uint32(sn)) & jnp.uint32(0xFF), jnp.int32)
                    plsc.addupdate_scatter(hist, [(b << 4) + lane], ones,
                                           mask=match)
                    mi = lax.convert_element_type(match, jnp.int32)
                    pos = w + plsc.cumsum(mi) - 1
                    plsc.store_scatter(ckey, [pos], plsc.bitcast(k, jnp.int32),
                                       mask=match)
                    w = w + plsc.all_reduce_population_count(match)
                return w
            w2 = lax.fori_loop(0, nb, pp, zeros16)
            bn, r2 = find_bucket(r)
            return w2[0], r2, bn

        m, r, b3 = refine(m, r, b2, 16, 8)
        m, r, b4 = refine(m, r, b3, 8, 0)

        b2u = lax.convert_element_type(b2, jnp.uint32)
        b3u = lax.convert_element_type(b3, jnp.uint32)
        b4u = lax.convert_element_type(b4, jnp.uint32)
        t_key = ((b1u << jnp.uint32(24)) | (b2u << jnp.uint32(16))
                 | (b3u << jnp.uint32(8)) | b4u)
        e_take = r

        if out_dma is not None:
            out_dma.wait()

        @plsc.parallel_loop(0, _NB, unroll=4, carry=zeros16)
        def _po(i, cnt):
            vf = row_buf[pl.ds(i * _L, _L)]
            k = _keyify(plsc.bitcast(vf, jnp.uint32))
            gt = k > t_key
            eq = k == t_key
            rank = cnt + plsc.cumsum(lax.convert_element_type(eq, jnp.int32))
            take = eq & (rank <= e_take)
            out_buf[pl.ds(i * _L, _L)] = jnp.where(
                gt | take, vf, jnp.float32(0.0))
            return cnt + plsc.all_reduce_population_count(eq)
        out_dma = pltpu.async_copy(out_buf, o_ref.at[row], sem)
    out_dma.wait()


def kernel(x):
    return _sc_topk(x)


# speedup vs baseline: 8.0836x; 1.1262x over previous
import functools

import jax
import jax.numpy as jnp
from jax import lax
from jax.experimental import pallas as pl
from jax.experimental.pallas import tpu as pltpu
from jax.experimental.pallas import tpu_sc as plsc

_K = 64
_ROWS = 64
_N = 32768
_L = 16
_NB = _N // _L
_NC = 2
_NS = 16
_NW = _NC * _NS
_RPW = _ROWS // _NW
_U = 4
_CAP = 8192


_mesh = plsc.VectorSubcoreMesh(
    core_axis_name="c", subcore_axis_name="s", num_cores=_NC, num_subcores=_NS)


def _keyify(v):
    s = v >> jnp.uint32(31)
    return v ^ ((jnp.uint32(0) - s) | jnp.uint32(0x80000000))


@functools.partial(
    pl.kernel,
    out_type=jax.ShapeDtypeStruct((_ROWS, _N), jnp.float32),
    mesh=_mesh,
    scratch_types=[
        pltpu.VMEM((_N,), jnp.float32),
        pltpu.VMEM((_N,), jnp.float32),
        pltpu.VMEM((_N + 2 * _L,), jnp.int32),
        pltpu.VMEM((_CAP + 2 * _L,), jnp.int32),
        pltpu.VMEM((264 * _L,), jnp.int32),
        pltpu.SemaphoreType.DMA,
    ],
    compiler_params=pltpu.CompilerParams(needs_layout_passes=False),
)
def _sc_topk(x_ref, o_ref, row_buf, out_buf, ckey, cidx, hist, sem):
    wid = lax.axis_index("s") * _NC + lax.axis_index("c")
    lane = lax.broadcasted_iota(jnp.int32, (_L,), 0)
    ones = jnp.ones((_L,), jnp.int32)
    zeros16 = jnp.zeros((_L,), jnp.int32)

    def zero_hist():
        @plsc.parallel_loop(0, 264, unroll=8)
        def _(i):
            hist[pl.ds(i * _L, _L)] = zeros16

    def find_bucket(r):
        @plsc.parallel_loop(0, 256, unroll=4, carry=zeros16)
        def _sfx(j, acc):
            b = 255 - j
            acc = acc + hist[pl.ds(b * _L, _L)]
            hist[pl.ds(b * _L, _L)] = acc
            return acc
        bkt = jnp.int32(0)
        for step in (128, 64, 32, 16, 8, 4, 2, 1):
            cand = bkt + step
            s = jnp.sum(hist[pl.ds(cand * _L, _L)])
            bkt = jnp.where(s >= r, cand, bkt)
        above = jnp.sum(hist[pl.ds((bkt + 1) * _L, _L)])
        return bkt, r - above

    out_dma = None
    for rr in range(_RPW):
        row = wid * _RPW + rr
        pltpu.sync_copy(x_ref.at[row], row_buf)

        zero_hist()

        @plsc.parallel_loop(0, _NB, unroll=8)
        def _p1(i):
            k = _keyify(plsc.bitcast(row_buf[pl.ds(i * _L, _L)], jnp.uint32))
            b = lax.convert_element_type(k >> jnp.uint32(24), jnp.int32)
            plsc.addupdate_scatter(hist, [(b << 4) + lane], ones)
        b1, r = find_bucket(jnp.int32(_K))
        b1u = lax.convert_element_type(b1, jnp.uint32)

        if out_dma is not None:
            out_dma.wait()

        @plsc.parallel_loop(0, _NB, unroll=4, carry=zeros16)
        def _p2(i, w):
            vf = row_buf[pl.ds(i * _L, _L)]
            k = _keyify(plsc.bitcast(vf, jnp.uint32))
            byte1 = k >> jnp.uint32(24)
            out_buf[pl.ds(i * _L, _L)] = jnp.where(byte1 > b1u, vf,
                                                   jnp.float32(0.0))
            match = byte1 == b1u
            mi = lax.convert_element_type(match, jnp.int32)
            pos = w + plsc.cumsum(mi) - 1
            plsc.store_scatter(ckey, [pos], plsc.bitcast(k, jnp.int32),
                               mask=match)
            plsc.store_scatter(cidx, [pos], i * _L + lane,
                               mask=match & (pos < _CAP))
            return w + plsc.all_reduce_population_count(match)
        m1 = _p2[0]

        nbc = (m1 + (_U * _L - 1)) // (_U * _L)

        def refine(r, pref_and_shift):
            zero_hist()
            *pref, sn = pref_and_shift

            def pp(i, _):
                for u in range(_U):
                    e0 = (i * _U + u) * _L
                    k = plsc.bitcast(ckey[pl.ds(e0, _L)], jnp.uint32)
                    match = (e0 + lane) < m1
                    for sp, bpu in pref:
                        match = match & (
                            ((k >> jnp.uint32(sp)) & jnp.uint32(0xFF)) == bpu)
                    b = lax.convert_element_type(
                        (k >> jnp.uint32(sn)) & jnp.uint32(0xFF), jnp.int32)
                    plsc.addupdate_scatter(hist, [(b << 4) + lane], ones,
                                           mask=match)
                return 0
            lax.fori_loop(0, nbc, pp, 0)
            bn, r2 = find_bucket(r)
            return bn, lax.convert_element_type(bn, jnp.uint32), r2

        b2, b2u, r = refine(r, (16,))
        b3, b3u, r = refine(r, ((16, b2u), 8))
        b4, b4u, r = refine(r, ((16, b2u), (8, b3u), 0))

        t_key = ((b1u << jnp.uint32(24)) | (b2u << jnp.uint32(16))
                 | (b3u << jnp.uint32(8)) | b4u)
        e_take = r

        def fast_fix(_):
            nbf = (m1 + (_L - 1)) // _L

            def fb(i, cnt):
                k = plsc.bitcast(ckey[pl.ds(i * _L, _L)], jnp.uint32)
                valid = (i * _L + lane) < m1
                p = cidx[pl.ds(i * _L, _L)]
                gt = valid & (k > t_key)
                eq = valid & (k == t_key)
                rank = cnt + plsc.cumsum(lax.convert_element_type(eq, jnp.int32))
                win = gt | (eq & (rank <= e_take))
                vf = plsc.load_gather(row_buf, [p], mask=valid)
                plsc.store_scatter(out_buf, [p], vf, mask=win)
                return cnt + plsc.all_reduce_population_count(eq)
            lax.fori_loop(0, nbf, fb, zeros16)
            return 0

        def slow_fix(_):
            @plsc.parallel_loop(0, _NB, unroll=4, carry=zeros16)
            def _po(i, cnt):
                vf = row_buf[pl.ds(i * _L, _L)]
                k = _keyify(plsc.bitcast(vf, jnp.uint32))
                gt = k > t_key
                eq = k == t_key
                rank = cnt + plsc.cumsum(lax.convert_element_type(eq, jnp.int32))
                take = eq & (rank <= e_take)
                out_buf[pl.ds(i * _L, _L)] = jnp.where(gt | take, vf,
                                                       jnp.float32(0.0))
                return cnt + plsc.all_reduce_population_count(eq)
            return 0

        lax.cond(m1 <= _CAP, fast_fix, slow_fix, 0)

        out_dma = pltpu.async_copy(out_buf, o_ref.at[row], sem)
    out_dma.wait()


def kernel(x):
    return _sc_topk(x)
